# Initial kernel scaffold; baseline (speedup 1.0000x reference)
import jax
import jax.numpy as jnp
from jax.experimental import pallas as pl


def kernel(x, edge_index, edge_attr, Wg, bg, Wf, Wa, ba):
    raise NotImplementedError("write your pallas kernel here")


# speedup vs baseline: 4.5076x; 4.5076x over previous
import functools

import jax
import jax.numpy as jnp
from jax import lax
from jax.experimental import pallas as pl
from jax.experimental.pallas import tpu as pltpu
from jax.experimental.pallas import tpu_sc as plsc

NC = 2
NS = 16
NW = NC * NS

DZS = 160
DGD = 32
DM = 144


def _node_tables_body(x_ref, wf_ref, wg_ref, bg_ref, a1_ref, a2_ref,
                      barow_ref, zs_ref, gd_ref):
    z = jnp.dot(x_ref[...], wf_ref[...], preferred_element_type=jnp.float32)
    u1 = z * a1_ref[...]
    u2 = z * a2_ref[...]
    dn = (((1,), (1,)), ((), ()))
    p = lax.dot_general(u1, wg_ref[...], dn,
                        preferred_element_type=jnp.float32)
    q = lax.dot_general(u2, wg_ref[...], dn,
                        preferred_element_type=jnp.float32)
    s = lax.dot_general(u1, bg_ref[...], dn,
                        preferred_element_type=jnp.float32)
    t = lax.dot_general(u2, bg_ref[...], dn,
                        preferred_element_type=jnp.float32)
    r = z.shape[0]
    pad15 = jnp.zeros((r, 15), jnp.float32)
    zs_ref[...] = jnp.concatenate([z, q, t, pad15], axis=1)
    gd_ref[...] = jnp.concatenate([p, s, pad15], axis=1) + barow_ref[...]


def _node_tables(x, Wf, Wg, bg, a1, a2, barow, n, r):
    grid = (n // r,)
    return pl.pallas_call(
        _node_tables_body,
        grid=grid,
        in_specs=[
            pl.BlockSpec((r, 128), lambda i: (i, 0)),
            pl.BlockSpec((128, 128), lambda i: (0, 0)),
            pl.BlockSpec((16, 128), lambda i: (0, 0)),
            pl.BlockSpec((1, 128), lambda i: (0, 0)),
            pl.BlockSpec((1, 128), lambda i: (0, 0)),
            pl.BlockSpec((1, 128), lambda i: (0, 0)),
            pl.BlockSpec((1, DGD), lambda i: (0, 0)),
        ],
        out_specs=[
            pl.BlockSpec((r, DZS), lambda i: (i, 0)),
            pl.BlockSpec((r, DGD), lambda i: (i, 0)),
        ],
        out_shape=[
            jax.ShapeDtypeStruct((n, DZS), jnp.float32),
            jax.ShapeDtypeStruct((n, DGD), jnp.float32),
        ],
    )(x, Wf, Wg, bg, a1, a2, barow)


def _make_gather_kernel(e, n, chunk):
    ep = e // NW
    nchunks = ep // chunk
    mesh = plsc.VectorSubcoreMesh(core_axis_name="c", subcore_axis_name="s")

    @functools.partial(
        pl.kernel,
        out_type=(
            jax.ShapeDtypeStruct((e, DZS), jnp.float32),
            jax.ShapeDtypeStruct((e, DGD), jnp.float32),
        ),
        mesh=mesh,
        scratch_types=[
            pltpu.VMEM((chunk,), jnp.int32),
            pltpu.VMEM((chunk,), jnp.int32),
            pltpu.VMEM((chunk, DZS), jnp.float32),
            pltpu.VMEM((chunk, DGD), jnp.float32),
            pltpu.SemaphoreType.DMA,
            pltpu.SemaphoreType.DMA,
        ],
        compiler_params=pltpu.CompilerParams(use_tc_tiling_on_sc=False),
    )
    def gather_kernel(src_hbm, dst_hbm, zs_hbm, gd_hbm, zsg_out, gdg_out,
                      src_v, dst_v, zs_v, gd_v, sem1, sem2):
        wid = lax.axis_index("s") * NC + lax.axis_index("c")
        tile_base = wid * ep

        def body(ci, carry):
            base = tile_base + ci * chunk
            pltpu.sync_copy(src_hbm.at[pl.ds(base, chunk)], src_v)
            pltpu.sync_copy(dst_hbm.at[pl.ds(base, chunk)], dst_v)
            cp1 = pltpu.async_copy(zs_hbm.at[src_v], zs_v, sem1)
            cp2 = pltpu.async_copy(gd_hbm.at[dst_v], gd_v, sem2)
            cp1.wait()
            cp2.wait()
            pltpu.sync_copy(zs_v, zsg_out.at[pl.ds(base, chunk)])
            pltpu.sync_copy(gd_v, gdg_out.at[pl.ds(base, chunk)])
            return carry

        lax.fori_loop(0, nchunks, body, 0)

    return gather_kernel


def _edge_math_body(ea_ref, zsg_ref, gdg_ref, wg_ref, bg_ref, m_ref):
    ea = ea_ref[...]
    k = jnp.dot(ea, wg_ref[...],
                preferred_element_type=jnp.float32) + bg_ref[...]
    pq = gdg_ref[:, 0:16] + zsg_ref[:, 128:144]
    e_raw = (jnp.sum(ea * pq, axis=1, keepdims=True)
             + gdg_ref[:, 16:17] + zsg_ref[:, 144:145])
    ex = jnp.exp(e_raw)
    w = jnp.where(e_raw > 0, ex, jnp.exp(ex - 1.0))
    m = w * k * zsg_ref[:, 0:128]
    b = m.shape[0]
    m_ref[...] = jnp.concatenate([m, w, jnp.zeros((b, 15), jnp.float32)],
                                 axis=1)


def _edge_math(ea, zsg, gdg, Wg, bg, e, beblk):
    grid = (e // beblk,)
    return pl.pallas_call(
        _edge_math_body,
        grid=grid,
        in_specs=[
            pl.BlockSpec((beblk, 16), lambda i: (i, 0)),
            pl.BlockSpec((beblk, DZS), lambda i: (i, 0)),
            pl.BlockSpec((beblk, DGD), lambda i: (i, 0)),
            pl.BlockSpec((16, 128), lambda i: (0, 0)),
            pl.BlockSpec((1, 128), lambda i: (0, 0)),
        ],
        out_specs=pl.BlockSpec((beblk, DM), lambda i: (i, 0)),
        out_shape=jax.ShapeDtypeStruct((e, DM), jnp.float32),
    )(ea, zsg, gdg, Wg, bg)


def _make_scatter_kernel(e, n, chunk):
    ec = e // NC
    ep = ec // NS
    nchunks = ep // chunk
    rows_per_tile = n // NS
    mesh = plsc.VectorSubcoreMesh(core_axis_name="c", subcore_axis_name="s")

    @functools.partial(
        pl.kernel,
        out_type=jax.ShapeDtypeStruct((NC, n, DM), jnp.float32),
        mesh=mesh,
        scratch_types=[
            pltpu.VMEM((chunk,), jnp.int32),
            pltpu.VMEM((chunk, DM), jnp.float32),
            pltpu.VMEM_SHARED((n, DM), jnp.float32),
            pltpu.SemaphoreType.DMA,
        ],
        compiler_params=pltpu.CompilerParams(use_tc_tiling_on_sc=False),
    )
    def scatter_kernel(dst_hbm, m_hbm, zeros_hbm, out_hbm,
                       dst_v, m_v, h_sh, sem):
        cid = lax.axis_index("c")
        sid = lax.axis_index("s")
        r0 = sid * rows_per_tile
        pltpu.sync_copy(zeros_hbm, h_sh.at[pl.ds(r0, rows_per_tile)])
        plsc.subcore_barrier()

        tile_base = cid * ec + sid * ep

        def body(ci, carry):
            base = tile_base + ci * chunk
            pltpu.sync_copy(dst_hbm.at[pl.ds(base, chunk)], dst_v)
            cp = pltpu.async_copy(m_hbm.at[pl.ds(base, chunk)], m_v, sem)
            cp.wait()
            pltpu.sync_copy(m_v, h_sh.at[dst_v], add=True)
            return carry

        lax.fori_loop(0, nchunks, body, 0)
        plsc.subcore_barrier()
        pltpu.sync_copy(h_sh.at[pl.ds(r0, rows_per_tile)],
                        out_hbm.at[cid, pl.ds(r0, rows_per_tile)])

    return scatter_kernel


def _norm_body(h0_ref, h1_ref, out_ref):
    h = h0_ref[...] + h1_ref[...]
    denom = jnp.maximum(h[:, 128:129], 1e-16)
    out_ref[...] = h[:, 0:128] / denom


def _normalize(hacc, n, r):
    grid = (n // r,)
    return pl.pallas_call(
        _norm_body,
        grid=grid,
        in_specs=[
            pl.BlockSpec((r, DM), lambda i: (i, 0)),
            pl.BlockSpec((r, DM), lambda i: (i, 0)),
        ],
        out_specs=pl.BlockSpec((r, 128), lambda i: (i, 0)),
        out_shape=jax.ShapeDtypeStruct((n, 128), jnp.float32),
    )(hacc[0], hacc[1])


def kernel(x, edge_index, edge_attr, Wg, bg, Wf, Wa, ba):
    n = x.shape[0]
    e = edge_index.shape[1]

    src = edge_index[0]
    dst = edge_index[1]
    a1 = Wa[0:128, 0].reshape(1, 128)
    a2 = Wa[128:256, 0].reshape(1, 128)
    bg2 = bg.reshape(1, 128)
    barow = jnp.zeros((1, DGD), jnp.float32).at[0, 16].set(ba[0])

    zs_tab, gd_tab = _node_tables(x, Wf, Wg, bg2, a1, a2, barow, n, r=1000)

    gather = _make_gather_kernel(e, n, chunk=80)
    zsg, gdg = gather(src, dst, zs_tab, gd_tab)

    m_ext = _edge_math(edge_attr, zsg, gdg, Wg, bg2, e, beblk=4000)

    zeros_block = jnp.zeros((n // NS, DM), jnp.float32)
    scatter = _make_scatter_kernel(e, n, chunk=80)
    hacc = scatter(dst, m_ext, zeros_block)

    return _normalize(hacc, n, r=1000)


# speedup vs baseline: 5.6380x; 1.2508x over previous
import functools

import jax
import jax.numpy as jnp
from jax import lax
from jax.experimental import pallas as pl
from jax.experimental.pallas import tpu as pltpu
from jax.experimental.pallas import tpu_sc as plsc

NC = 2
NS = 16
NW = NC * NS
L = 16

DZS = 160
DGD = 32
DM = 144


def _node_tables_body(x_ref, wf_ref, wg_ref, bg_ref, a1_ref, a2_ref,
                      barow_ref, zs_ref, gd_ref):
    z = jnp.dot(x_ref[...], wf_ref[...], preferred_element_type=jnp.float32)
    u1 = z * a1_ref[...]
    u2 = z * a2_ref[...]
    dn = (((1,), (1,)), ((), ()))
    p = lax.dot_general(u1, wg_ref[...], dn,
                        preferred_element_type=jnp.float32)
    q = lax.dot_general(u2, wg_ref[...], dn,
                        preferred_element_type=jnp.float32)
    s = lax.dot_general(u1, bg_ref[...], dn,
                        preferred_element_type=jnp.float32)
    t = lax.dot_general(u2, bg_ref[...], dn,
                        preferred_element_type=jnp.float32)
    r = z.shape[0]
    pad15 = jnp.zeros((r, 15), jnp.float32)
    zs_ref[...] = jnp.concatenate([z, q, t, pad15], axis=1)
    gd_ref[...] = jnp.concatenate([p, s, pad15], axis=1) + barow_ref[...]


def _node_tables(x, Wf, Wg, bg, a1, a2, barow, n, r):
    grid = (n // r,)
    return pl.pallas_call(
        _node_tables_body,
        grid=grid,
        in_specs=[
            pl.BlockSpec((r, 128), lambda i: (i, 0)),
            pl.BlockSpec((128, 128), lambda i: (0, 0)),
            pl.BlockSpec((16, 128), lambda i: (0, 0)),
            pl.BlockSpec((1, 128), lambda i: (0, 0)),
            pl.BlockSpec((1, 128), lambda i: (0, 0)),
            pl.BlockSpec((1, 128), lambda i: (0, 0)),
            pl.BlockSpec((1, DGD), lambda i: (0, 0)),
        ],
        out_specs=[
            pl.BlockSpec((r, DZS), lambda i: (i, 0)),
            pl.BlockSpec((r, DGD), lambda i: (i, 0)),
        ],
        out_shape=[
            jax.ShapeDtypeStruct((n, DZS), jnp.float32),
            jax.ShapeDtypeStruct((n, DGD), jnp.float32),
        ],
    )(x, Wf, Wg, bg, a1, a2, barow)


def _k_matmul_body(ea_ref, wg_ref, bg_ref, k_ref):
    k_ref[...] = jnp.dot(ea_ref[...], wg_ref[...],
                         preferred_element_type=jnp.float32) + bg_ref[...]


def _k_matmul(ea, Wg, bg, e, beblk):
    grid = (e // beblk,)
    return pl.pallas_call(
        _k_matmul_body,
        grid=grid,
        in_specs=[
            pl.BlockSpec((beblk, 16), lambda i: (i, 0)),
            pl.BlockSpec((16, 128), lambda i: (0, 0)),
            pl.BlockSpec((1, 128), lambda i: (0, 0)),
        ],
        out_specs=pl.BlockSpec((beblk, 128), lambda i: (i, 0)),
        out_shape=jax.ShapeDtypeStruct((e, 128), jnp.float32),
    )(ea, Wg, bg)


def _make_edge_kernel(e, n, chunk):
    ec = e // NC
    ep = ec // NS
    nchunks = ep // chunk
    ngroups = chunk // L
    rows_per_tile = n // NS
    mesh = plsc.VectorSubcoreMesh(core_axis_name="c", subcore_axis_name="s")

    @functools.partial(
        pl.kernel,
        out_type=jax.ShapeDtypeStruct((NC, n, DM), jnp.float32),
        mesh=mesh,
        scratch_types=[
            pltpu.VMEM((chunk,), jnp.int32),
            pltpu.VMEM((chunk,), jnp.int32),
            pltpu.VMEM((chunk, 16), jnp.float32),
            pltpu.VMEM((chunk, DGD), jnp.float32),
            pltpu.VMEM((chunk, DZS), jnp.float32),
            pltpu.VMEM((chunk, 128), jnp.float32),
            pltpu.VMEM((chunk, DM), jnp.float32),
            pltpu.VMEM((chunk,), jnp.float32),
            pltpu.VMEM_SHARED((n, DM), jnp.float32),
            pltpu.SemaphoreType.DMA,
            pltpu.SemaphoreType.DMA,
            pltpu.SemaphoreType.DMA,
            pltpu.SemaphoreType.DMA,
        ],
        compiler_params=pltpu.CompilerParams(use_tc_tiling_on_sc=False,
                                             needs_layout_passes=False),
    )
    def edge_kernel(src_hbm, dst_hbm, ea_hbm, zs_hbm, gd_hbm, k_hbm,
                    zeros_hbm, out_hbm,
                    src_v, dst_v, ea_v, gd_v, zs_v, k_v, m_v, w_v, h_sh,
                    sem1, sem2, sem3, sem4):
        cid = lax.axis_index("c")
        sid = lax.axis_index("s")
        r0 = sid * rows_per_tile
        pltpu.sync_copy(zeros_hbm, h_sh.at[pl.ds(r0, rows_per_tile)])
        plsc.subcore_barrier()

        tile_base = cid * ec + sid * ep
        lanes = lax.iota(jnp.int32, L)

        def chunk_body(ci, carry):
            base = tile_base + ci * chunk
            pltpu.sync_copy(src_hbm.at[pl.ds(base, chunk)], src_v)
            pltpu.sync_copy(dst_hbm.at[pl.ds(base, chunk)], dst_v)
            cp1 = pltpu.async_copy(zs_hbm.at[src_v], zs_v, sem1)
            cp2 = pltpu.async_copy(gd_hbm.at[dst_v], gd_v, sem2)
            cp3 = pltpu.async_copy(ea_hbm.at[pl.ds(base, chunk)], ea_v, sem3)
            cp4 = pltpu.async_copy(k_hbm.at[pl.ds(base, chunk)], k_v, sem4)
            cp3.wait()
            cp2.wait()
            cp1.wait()
            cp4.wait()

            def grp_body(g, carry2):
                rowi = lanes + g * L
                acc = jnp.zeros((L,), jnp.float32)
                for c in range(16):
                    col = jnp.full((L,), c, jnp.int32)
                    eac = plsc.load_gather(ea_v, [rowi, col])
                    pc = plsc.load_gather(gd_v, [rowi, col])
                    qc = plsc.load_gather(zs_v, [rowi, col + 128])
                    acc = acc + eac * (pc + qc)
                sv = plsc.load_gather(gd_v, [rowi, jnp.full((L,), 16,
                                                            jnp.int32)])
                tv = plsc.load_gather(zs_v, [rowi, jnp.full((L,), 144,
                                                            jnp.int32)])
                er = acc + sv + tv
                ex = jnp.exp(er)
                w = jnp.where(er > 0, ex, jnp.exp(ex - 1.0))
                w_v[pl.ds(g * L, L)] = w
                return carry2

            lax.fori_loop(0, ngroups, grp_body, 0)

            def edge_body(i, carry2):
                wb = plsc.load_gather(w_v, [jnp.full((L,), i, jnp.int32)])
                for j in range(8):
                    m_v[i, pl.ds(j * L, L)] = (
                        wb * k_v[i, pl.ds(j * L, L)]
                        * zs_v[i, pl.ds(j * L, L)])
                m_v[i, pl.ds(128, L)] = jnp.where(lanes == 0, wb, 0.0)
                return carry2

            lax.fori_loop(0, chunk, edge_body, 0)

            pltpu.sync_copy(m_v, h_sh.at[dst_v], add=True)
            return carry

        lax.fori_loop(0, nchunks, chunk_body, 0)
        plsc.subcore_barrier()
        pltpu.sync_copy(h_sh.at[pl.ds(r0, rows_per_tile)],
                        out_hbm.at[cid, pl.ds(r0, rows_per_tile)])

    return edge_kernel


def _norm_body(h0_ref, h1_ref, out_ref):
    h = h0_ref[...] + h1_ref[...]
    denom = jnp.maximum(h[:, 128:129], 1e-16)
    out_ref[...] = h[:, 0:128] / denom


def _normalize(hacc, n, r):
    grid = (n // r,)
    return pl.pallas_call(
        _norm_body,
        grid=grid,
        in_specs=[
            pl.BlockSpec((r, DM), lambda i: (i, 0)),
            pl.BlockSpec((r, DM), lambda i: (i, 0)),
        ],
        out_specs=pl.BlockSpec((r, 128), lambda i: (i, 0)),
        out_shape=jax.ShapeDtypeStruct((n, 128), jnp.float32),
    )(hacc[0], hacc[1])


def kernel(x, edge_index, edge_attr, Wg, bg, Wf, Wa, ba):
    n = x.shape[0]
    e = edge_index.shape[1]

    src = edge_index[0]
    dst = edge_index[1]
    a1 = Wa[0:128, 0].reshape(1, 128)
    a2 = Wa[128:256, 0].reshape(1, 128)
    bg2 = bg.reshape(1, 128)
    barow = jnp.zeros((1, DGD), jnp.float32).at[0, 16].set(ba[0])

    zs_tab, gd_tab = _node_tables(x, Wf, Wg, bg2, a1, a2, barow, n, r=1000)
    k_tab = _k_matmul(edge_attr, Wg, bg2, e, beblk=4000)

    zeros_block = jnp.zeros((n // NS, DM), jnp.float32)
    edge_k = _make_edge_kernel(e, n, chunk=80)
    hacc = edge_k(src, dst, edge_attr, zs_tab, gd_tab, k_tab, zeros_block)

    return _normalize(hacc, n, r=1000)
